# baseline (speedup 1.0000x reference)
import functools

import jax
import jax.numpy as jnp
from jax import lax
from jax.experimental import pallas as pl
from jax.experimental.pallas import tpu as pltpu
from jax.experimental.pallas import tpu_sc as plsc


def _lookup_kernel(NI, NJ, D, CH):
    info = plsc.get_sparse_core_info()
    NC, NS = info.num_cores, info.num_subcores
    NW = NC * NS
    NB_I = NI // CH
    per_w = NB_I // NW
    n_sub = per_w * NJ
    n_pairs = n_sub // 2
    W = CH * NJ
    assert NI % CH == 0 and NB_I % NW == 0 and NJ % 2 == 0
    mesh = plsc.VectorSubcoreMesh(core_axis_name="c", subcore_axis_name="s")

    @functools.partial(
        pl.kernel,
        out_type=jax.ShapeDtypeStruct((NJ, D, NI), jnp.float32),
        mesh=mesh,
        scratch_types=[
            pltpu.VMEM((W,), jnp.int32),
            pltpu.VMEM((CH,), jnp.int32),
            pltpu.VMEM((CH,), jnp.int32),
            pltpu.VMEM((CH,), jnp.int32),
            pltpu.VMEM((CH,), jnp.int32),
            pltpu.VMEM((2, CH, 2 * D), jnp.float32),
            pltpu.VMEM((2, D, CH), jnp.float32),
            pltpu.SemaphoreType.DMA,
            pltpu.SemaphoreType.DMA,
            pltpu.SemaphoreType.DMA,
            pltpu.SemaphoreType.DMA,
        ],
        compiler_params=pltpu.CompilerParams(
            use_tc_tiling_on_sc=False, needs_layout_passes=False),
    )
    def k(idx1d, t128, out, win, p0, p1, h0, h1, grows, oblk, g0, g1, s0, s1):
        wid = lax.axis_index("s") * NC + lax.axis_index("c")
        prefs = (p0, p1)
        hrefs = (h0, h1)
        gsems = (g0, g1)
        ssems = (s0, s1)
        lanes = lax.iota(jnp.int32, 16)
        lanesj = lanes * NJ

        def decode(n):
            m = n // NJ
            j = n - m * NJ
            i0 = (wid + m * NW) * CH
            return j, i0

        def prep(n, s):
            j, i0 = decode(n)

            @pl.when(j == 0)
            def _():
                pltpu.sync_copy(idx1d.at[pl.ds(i0 * NJ, W)], win)

            pref, href = prefs[s], hrefs[s]
            for g in range(CH // 16):
                v = plsc.load_gather(win, [lanesj + (g * 16 * NJ + j)])
                pref[pl.ds(g * 16, 16)] = lax.shift_right_logical(v, 1)
                href[pl.ds(g * 16, 16)] = (v & 1) * D
            pltpu.async_copy(t128.at[pref], grows.at[s], gsems[s])

        def gather_wait(s):
            pltpu.make_async_copy(
                t128.at[prefs[s]], grows.at[s], gsems[s]).wait()

        def store_desc(n, s):
            j, i0 = decode(n)
            return pltpu.make_async_copy(
                oblk.at[s], out.at[j, :, pl.ds(i0, CH)], ssems[s])

        def transpose(s):
            gref = grows.at[s]
            oref = oblk.at[s]
            href = hrefs[s]

            @pl.loop(0, CH // 16)
            def _(g):
                row = lanes + g * 16
                colb = href[pl.ds(g * 16, 16)]
                for d in range(D):
                    vals = plsc.load_gather(gref, [row, colb + d])
                    oref[d, pl.ds(g * 16, 16)] = vals

        prep(0, 0)

        @pl.loop(0, n_pairs)
        def _(h):
            n0 = 2 * h
            prep(n0 + 1, 1)
            gather_wait(0)

            @pl.when(h > 0)
            def _():
                store_desc(n0 - 2, 0).wait()

            transpose(0)
            store_desc(n0, 0).start()

            @pl.when(h < n_pairs - 1)
            def _():
                prep(n0 + 2, 0)

            gather_wait(1)

            @pl.when(h > 0)
            def _():
                store_desc(n0 - 1, 1).wait()

            transpose(1)
            store_desc(n0 + 1, 1).start()

        store_desc(n_sub - 2, 0).wait()
        store_desc(n_sub - 1, 1).wait()

    return k


@jax.jit
def kernel(data, table):
    NI, NJ = data.shape
    V, D = table.shape
    t128 = table.reshape(V // 2, 2 * D)
    idx1d = data.reshape(NI * NJ)
    out_phys = _lookup_kernel(NI, NJ, D, 128)(idx1d, t128)
    return out_phys.transpose(2, 0, 1)


# speedup vs baseline: 1.1127x; 1.1127x over previous
import functools

import jax
import jax.numpy as jnp
from jax import lax
from jax.experimental import pallas as pl
from jax.experimental.pallas import tpu as pltpu
from jax.experimental.pallas import tpu_sc as plsc


def _lookup_kernel(NI, NJ, D, CH):
    info = plsc.get_sparse_core_info()
    NC, NS = info.num_cores, info.num_subcores
    NW = NC * NS
    NB_I = NI // CH
    per_w = NB_I // NW
    n_sub = per_w * NJ
    n_pairs = n_sub // 2
    W = CH * NJ
    assert NI % CH == 0 and NB_I % NW == 0 and NJ % 2 == 0
    mesh = plsc.VectorSubcoreMesh(core_axis_name="c", subcore_axis_name="s")

    @functools.partial(
        pl.kernel,
        out_type=jax.ShapeDtypeStruct((NJ, D // 8, NI // CH, 8, CH),
                                      jnp.float32),
        mesh=mesh,
        scratch_types=[
            pltpu.VMEM((W,), jnp.int32),
            pltpu.VMEM((CH,), jnp.int32),
            pltpu.VMEM((CH,), jnp.int32),
            pltpu.VMEM((CH,), jnp.int32),
            pltpu.VMEM((CH,), jnp.int32),
            pltpu.VMEM((2, CH, 2 * D), jnp.float32),
            pltpu.VMEM((2, D, CH), jnp.float32),
            pltpu.SemaphoreType.DMA,
            pltpu.SemaphoreType.DMA,
            pltpu.SemaphoreType.DMA,
            pltpu.SemaphoreType.DMA,
        ],
        compiler_params=pltpu.CompilerParams(
            use_tc_tiling_on_sc=False, needs_layout_passes=False),
    )
    def k(idx1d, t128, out, win, p0, p1, h0, h1, grows, oblk, g0, g1, s0, s1):
        wid = lax.axis_index("s") * NC + lax.axis_index("c")
        prefs = (p0, p1)
        hrefs = (h0, h1)
        gsems = (g0, g1)
        ssems = (s0, s1)
        lanes = lax.iota(jnp.int32, 16)
        lanesj = lanes * NJ

        def decode(n):
            m = n // NJ
            j = n - m * NJ
            i0 = (wid + m * NW) * CH
            return j, i0

        def prep(n, s):
            j, i0 = decode(n)

            @pl.when(j == 0)
            def _():
                pltpu.sync_copy(idx1d.at[pl.ds(i0 * NJ, W)], win)

            pref, href = prefs[s], hrefs[s]
            for g in range(CH // 16):
                v = plsc.load_gather(win, [lanesj + (g * 16 * NJ + j)])
                pref[pl.ds(g * 16, 16)] = lax.shift_right_logical(v, 1)
                href[pl.ds(g * 16, 16)] = (v & 1) * D
            pltpu.async_copy(t128.at[pref], grows.at[s], gsems[s])

        def gather_wait(s):
            pltpu.make_async_copy(
                t128.at[prefs[s]], grows.at[s], gsems[s]).wait()

        def store_start(n, s):
            j, i0 = decode(n)
            ib = i0 // CH
            for tr in range(D // 8):
                pltpu.make_async_copy(
                    oblk.at[s, pl.ds(8 * tr, 8)], out.at[j, tr, ib],
                    ssems[s]).start()

        def store_wait(s):
            for tr in range(D // 8):
                pltpu.make_async_copy(
                    oblk.at[s, pl.ds(8 * tr, 8)], out.at[0, tr, 0],
                    ssems[s]).wait()

        def transpose(s):
            gref = grows.at[s]
            oref = oblk.at[s]
            href = hrefs[s]

            @pl.loop(0, CH // 16)
            def _(g):
                row = lanes + g * 16
                colb = href[pl.ds(g * 16, 16)]
                for d in range(D):
                    vals = plsc.load_gather(gref, [row, colb + d])
                    oref[d, pl.ds(g * 16, 16)] = vals

        prep(0, 0)

        @pl.loop(0, n_pairs)
        def _(h):
            n0 = 2 * h
            prep(n0 + 1, 1)
            gather_wait(0)

            @pl.when(h > 0)
            def _():
                store_wait(0)

            transpose(0)
            store_start(n0, 0)

            @pl.when(h < n_pairs - 1)
            def _():
                prep(n0 + 2, 0)

            gather_wait(1)

            @pl.when(h > 0)
            def _():
                store_wait(1)

            transpose(1)
            store_start(n0 + 1, 1)

        store_wait(0)
        store_wait(1)

    return k


@jax.jit
def kernel(data, table):
    NI, NJ = data.shape
    V, D = table.shape
    t128 = table.reshape(V // 2, 2 * D)
    idx1d = data.reshape(NI * NJ)
    CH = 128
    out_t = _lookup_kernel(NI, NJ, D, CH)(idx1d, t128)
    out_phys = out_t.transpose(0, 1, 3, 2, 4).reshape(NJ, D, NI)
    return out_phys.transpose(2, 0, 1)


# speedup vs baseline: 1.5382x; 1.3824x over previous
import functools

import jax
import jax.numpy as jnp
from jax import lax
from jax.experimental import pallas as pl
from jax.experimental import layout as jlayout
from jax.experimental.pallas import tpu as pltpu
from jax.experimental.pallas import tpu_sc as plsc


def _gather_kernel(B, D, CH, K):
    info = plsc.get_sparse_core_info()
    NC, NS = info.num_cores, info.num_subcores
    NW = NC * NS
    b_per_w = B // NW
    n_chunks = b_per_w // CH
    n_pairs = n_chunks // (2 * K)
    assert B % NW == 0 and b_per_w % CH == 0 and n_chunks % (2 * K) == 0
    mesh = plsc.VectorSubcoreMesh(core_axis_name="c", subcore_axis_name="s")

    @functools.partial(
        pl.kernel,
        out_type=jax.ShapeDtypeStruct((B, 2 * D), jnp.float32),
        mesh=mesh,
        scratch_types=[
            pltpu.VMEM((b_per_w,), jnp.int32),
            pltpu.VMEM((2 * K, CH, D), jnp.float32),
            pltpu.SemaphoreType.DMA,
            pltpu.SemaphoreType.DMA,
            pltpu.SemaphoreType.DMA,
            pltpu.SemaphoreType.DMA,
        ],
        compiler_params=pltpu.CompilerParams(use_tc_tiling_on_sc=False),
    )
    def k(idx_hbm, table_hbm, out_hbm, idx_all, rows, gsem_a, gsem_b, ssem_a, ssem_b):
        wid = lax.axis_index("s") * NC + lax.axis_index("c")
        base = wid * b_per_w
        pltpu.sync_copy(idx_hbm.at[pl.ds(base, b_per_w)], idx_all)

        def gather_desc(g, half, b, sem):
            ch = g * K + b
            idx_sl = idx_all.at[pl.ds(ch * CH, CH)]
            return pltpu.make_async_copy(
                table_hbm.at[idx_sl], rows.at[half * K + b], sem)

        def store_desc(g, half, b, sem):
            ch = g * K + b
            return pltpu.make_async_copy(
                rows.at[half * K + b],
                out_hbm.at[pl.ds(base + ch * CH, CH), pl.ds(0, D)], sem)

        def fire_gathers(g, half, sem):
            for b in range(K):
                gather_desc(g, half, b, sem).start()

        def drain_gathers(g, half, sem):
            for b in range(K):
                gather_desc(g, half, b, sem).wait()

        def fire_stores(g, half, sem):
            for b in range(K):
                store_desc(g, half, b, sem).start()

        def drain_stores(g, half, sem):
            for b in range(K):
                store_desc(g, half, b, sem).wait()

        @pl.loop(0, n_pairs)
        def _(h):
            g0 = 2 * h
            g1 = 2 * h + 1

            @pl.when(h > 0)
            def _():
                drain_stores(g0 - 2, 0, ssem_a)

            fire_gathers(g0, 0, gsem_a)
            drain_gathers(g0, 0, gsem_a)

            @pl.when(h > 0)
            def _():
                drain_stores(g1 - 2, 1, ssem_b)

            fire_gathers(g1, 1, gsem_b)
            fire_stores(g0, 0, ssem_a)
            drain_gathers(g1, 1, gsem_b)
            fire_stores(g1, 1, ssem_b)

        drain_stores(2 * n_pairs - 2, 0, ssem_a)
        drain_stores(2 * n_pairs - 1, 1, ssem_b)

    return k


@jax.jit
def kernel(data, table):
    B = data.size
    V, D = table.shape
    flat = data.reshape(B)
    out2 = _gather_kernel(B, D, 128, 4)(flat, table)
    return out2[:, :D].reshape(*data.shape, D)
